# baseline (speedup 1.0000x reference)
import dataclasses

import jax
import jax.numpy as jnp
from jax import lax
from jax.experimental import pallas as pl
from jax.experimental.pallas import tpu as pltpu
from jax.experimental.pallas import tpu_sc as plsc

_N1 = 8192
_N2 = 8192
_C = 16
_R = 256
_NBLK = _N1 // _R

_SC_SUBCORES = 16
_SC_LANES = 16
_SEG = _N2 // _SC_SUBCORES


def _sc_compiler_params():
    cp = pltpu.CompilerParams()
    if "needs_layout_passes" in pltpu.CompilerParams.__dataclass_fields__:
        cp = dataclasses.replace(cp, needs_layout_passes=False)
    return cp


def _tc_body(y2r_ref, f2_ref, f_ref, y2c_ref, y1c_ref, out_ref, acc_ref):
    i = pl.program_id(0)
    _CW = 512

    def chunked_sums(x_ref, rowlab):
        parts_m = []
        parts_t = []
        for c in range(_N2 // _CW):
            sl = slice(c * _CW, (c + 1) * _CW)
            xc = x_ref[:, sl]
            mc = rowlab == y2r_ref[:, sl]
            parts_m.append(jnp.sum(jnp.where(mc, xc, 0.0), axis=0, keepdims=True))
            parts_t.append(jnp.sum(xc, axis=0, keepdims=True))
        return (jnp.concatenate(parts_m, axis=1),
                jnp.concatenate(parts_t, axis=1))

    s_m1, s_t1 = chunked_sums(f2_ref, y2c_ref[...])
    s_m2, s_t2 = chunked_sums(f_ref, y1c_ref[...])

    @pl.when(i == 0)
    def _init():
        acc_ref[0] = s_m1
        acc_ref[1] = s_t1
        acc_ref[2] = s_m2
        acc_ref[3] = s_t2

    @pl.when(i > 0)
    def _accum():
        acc_ref[0] += s_m1
        acc_ref[1] += s_t1
        acc_ref[2] += s_m2
        acc_ref[3] += s_t2

    @pl.when(i == _NBLK - 1)
    def _finalize():
        out_ref[0] = jnp.sum(acc_ref[0])
        out_ref[1] = jnp.sum(acc_ref[1])
        out_ref[2] = jnp.sum(acc_ref[2])
        out_ref[3] = jnp.sum(acc_ref[3])
        for k in range(4, 16):
            out_ref[k] = 0.0


_NTILE = 32


def _sc_hist_body(labs_hbm, out_hbm):
    def tile_body(in_vmem, out_vmem):
        iota = lax.iota(jnp.int32, _SC_LANES)
        accs = tuple(jnp.zeros((_SC_LANES,), jnp.float32) for _ in range(_C))

        def chunk(j, accs):
            v = in_vmem[pl.ds(j * _SC_LANES, _SC_LANES)]
            return tuple(
                acc + jnp.where(v == c, 1.0, 0.0)
                for c, acc in enumerate(accs)
            )

        accs = lax.fori_loop(0, _SEG // _SC_LANES, chunk, accs)
        h = jnp.zeros((_SC_LANES,), jnp.float32)
        for c in range(_C):
            h = jnp.where(iota == c, jnp.sum(accs[c]), h)
        out_vmem[...] = h

    pltpu.emit_pipeline(
        tile_body,
        grid=(_NTILE,),
        in_specs=[pl.BlockSpec((_SEG,), lambda i: (i,))],
        out_specs=[pl.BlockSpec((_SC_LANES,), lambda i: (i,))],
        core_axis_name=("c", "s"),
        dimension_semantics=(pltpu.PARALLEL,),
    )(labs_hbm, out_hbm)


def _tc_combine_body(parts_ref, rows_ref, out_ref):
    h1 = jnp.sum(rows_ref[0:_SC_SUBCORES, :], axis=0)
    h2 = jnp.sum(rows_ref[_SC_SUBCORES:_NTILE, :], axis=0)
    c22 = jnp.sum(h2 * h2)
    c12 = jnp.sum(h1 * h2)
    m1 = parts_ref[0]
    t1 = parts_ref[1]
    m2 = parts_ref[2]
    t2 = parts_ref[3]
    n1 = t1 - m1
    n2 = t2 - m2
    same1 = c22 - jnp.float32(_N2)
    different1 = jnp.float32(_N2) * jnp.float32(_N2) - c22
    same2 = c12
    different2 = jnp.float32(_N1) * jnp.float32(_N2) - c12
    out_ref[0] = (m1 / same1 + m2 / same2) / (
        n1 / different1 + n2 / different2
    )


def kernel(y1, y2, f, f2):
    y1 = y1.astype(jnp.int32)
    y2 = y2.astype(jnp.int32)

    parts = pl.pallas_call(
        _tc_body,
        grid=(_NBLK,),
        in_specs=[
            pl.BlockSpec((1, _N2), lambda i: (0, 0)),
            pl.BlockSpec((_R, _N2), lambda i: (i, 0)),
            pl.BlockSpec((_R, _N2), lambda i: (i, 0)),
            pl.BlockSpec((_R, 1), lambda i: (i, 0)),
            pl.BlockSpec((_R, 1), lambda i: (i, 0)),
        ],
        out_specs=pl.BlockSpec(memory_space=pltpu.SMEM),
        out_shape=jax.ShapeDtypeStruct((16,), jnp.float32),
        scratch_shapes=[pltpu.VMEM((4, 1, _N2), jnp.float32)],
        compiler_params=pltpu.CompilerParams(
            dimension_semantics=("arbitrary",),
        ),
    )(
        y2.reshape(1, _N2),
        f2,
        f,
        y2.reshape(_N2, 1),
        y1.reshape(_N1, 1),
    )

    labs = jnp.concatenate([y1, y2])
    rows = pl.kernel(
        _sc_hist_body,
        out_type=jax.ShapeDtypeStruct((_NTILE * _SC_LANES,), jnp.float32),
        mesh=plsc.VectorSubcoreMesh(core_axis_name="c", subcore_axis_name="s"),
        scratch_types=[],
        compiler_params=_sc_compiler_params(),
    )(labs)

    res = pl.pallas_call(
        _tc_combine_body,
        in_specs=[
            pl.BlockSpec(memory_space=pltpu.SMEM),
            pl.BlockSpec(memory_space=pltpu.VMEM),
        ],
        out_specs=pl.BlockSpec(memory_space=pltpu.SMEM),
        out_shape=jax.ShapeDtypeStruct((1,), jnp.float32),
    )(parts, rows.reshape(_NTILE, _SC_LANES))

    return res[0]


# speedup vs baseline: 1.0013x; 1.0013x over previous
import dataclasses

import jax
import jax.numpy as jnp
from jax import lax
from jax.experimental import pallas as pl
from jax.experimental.pallas import tpu as pltpu
from jax.experimental.pallas import tpu_sc as plsc

_N1 = 8192
_N2 = 8192
_C = 16
_R = 256
_NBLK = _N1 // _R

_SC_SUBCORES = 16
_SC_LANES = 16
_SEG = _N2 // _SC_SUBCORES


def _sc_compiler_params():
    cp = pltpu.CompilerParams()
    if "needs_layout_passes" in pltpu.CompilerParams.__dataclass_fields__:
        cp = dataclasses.replace(cp, needs_layout_passes=False)
    return cp


def _tc_body(y2r_ref, f2_ref, f_ref, y2c_ref, y1c_ref, out_ref, acc_ref):
    i = pl.program_id(0)
    _CW = 512

    def chunked_sums(x_ref, rowlab):
        parts_m = []
        parts_t = []
        for c in range(_N2 // _CW):
            sl = slice(c * _CW, (c + 1) * _CW)
            xc = x_ref[:, sl]
            mc = rowlab == y2r_ref[:, sl]
            parts_m.append(jnp.sum(jnp.where(mc, xc, 0.0), axis=0, keepdims=True))
            parts_t.append(jnp.sum(xc, axis=0, keepdims=True))
        return (jnp.concatenate(parts_m, axis=1),
                jnp.concatenate(parts_t, axis=1))

    s_m1, s_t1 = chunked_sums(f2_ref, y2c_ref[...])
    s_m2, s_t2 = chunked_sums(f_ref, y1c_ref[...])

    @pl.when(i == 0)
    def _init():
        acc_ref[0] = s_m1
        acc_ref[1] = s_t1
        acc_ref[2] = s_m2
        acc_ref[3] = s_t2

    @pl.when(i > 0)
    def _accum():
        acc_ref[0] += s_m1
        acc_ref[1] += s_t1
        acc_ref[2] += s_m2
        acc_ref[3] += s_t2

    @pl.when(i == _NBLK - 1)
    def _finalize():
        out_ref[0] = jnp.sum(acc_ref[0])
        out_ref[1] = jnp.sum(acc_ref[1])
        out_ref[2] = jnp.sum(acc_ref[2])
        out_ref[3] = jnp.sum(acc_ref[3])
        for k in range(4, 16):
            out_ref[k] = 0.0


_NTILE = 32


def _sc_hist_body(y1_hbm, y2_hbm, out_hbm):
    def tile_body(in_vmem, out_vmem):
        iota = lax.iota(jnp.int32, _SC_LANES)
        accs = tuple(jnp.zeros((_SC_LANES,), jnp.float32) for _ in range(_C))

        def chunk(j, accs):
            v = in_vmem[pl.ds(j * _SC_LANES, _SC_LANES)]
            return tuple(
                acc + jnp.where(v == c, 1.0, 0.0)
                for c, acc in enumerate(accs)
            )

        accs = lax.fori_loop(0, _SEG // _SC_LANES, chunk, accs)
        h = jnp.zeros((_SC_LANES,), jnp.float32)
        for c in range(_C):
            h = jnp.where(iota == c, jnp.sum(accs[c]), h)
        out_vmem[...] = h

    for idx, src in enumerate((y1_hbm, y2_hbm)):
        pltpu.emit_pipeline(
            tile_body,
            grid=(_SC_SUBCORES,),
            in_specs=[pl.BlockSpec((_SEG,), lambda i: (i,))],
            out_specs=[
                pl.BlockSpec(
                    (_SC_LANES,),
                    lambda i, idx=idx: (idx * _SC_SUBCORES + i,),
                )
            ],
            core_axis_name=("c", "s"),
            dimension_semantics=(pltpu.PARALLEL,),
        )(src, out_hbm)


def _tc_combine_body(parts_ref, rows_ref, out_ref):
    h1 = jnp.sum(rows_ref[0:_SC_SUBCORES, :], axis=0)
    h2 = jnp.sum(rows_ref[_SC_SUBCORES:_NTILE, :], axis=0)
    c22 = jnp.sum(h2 * h2)
    c12 = jnp.sum(h1 * h2)
    m1 = parts_ref[0]
    t1 = parts_ref[1]
    m2 = parts_ref[2]
    t2 = parts_ref[3]
    n1 = t1 - m1
    n2 = t2 - m2
    same1 = c22 - jnp.float32(_N2)
    different1 = jnp.float32(_N2) * jnp.float32(_N2) - c22
    same2 = c12
    different2 = jnp.float32(_N1) * jnp.float32(_N2) - c12
    out_ref[0] = (m1 / same1 + m2 / same2) / (
        n1 / different1 + n2 / different2
    )


def kernel(y1, y2, f, f2):
    y1 = y1.astype(jnp.int32)
    y2 = y2.astype(jnp.int32)

    parts = pl.pallas_call(
        _tc_body,
        grid=(_NBLK,),
        in_specs=[
            pl.BlockSpec((1, _N2), lambda i: (0, 0)),
            pl.BlockSpec((_R, _N2), lambda i: (i, 0)),
            pl.BlockSpec((_R, _N2), lambda i: (i, 0)),
            pl.BlockSpec((_R, 1), lambda i: (i, 0)),
            pl.BlockSpec((_R, 1), lambda i: (i, 0)),
        ],
        out_specs=pl.BlockSpec(memory_space=pltpu.SMEM),
        out_shape=jax.ShapeDtypeStruct((16,), jnp.float32),
        scratch_shapes=[pltpu.VMEM((4, 1, _N2), jnp.float32)],
        compiler_params=pltpu.CompilerParams(
            dimension_semantics=("arbitrary",),
        ),
    )(
        y2.reshape(1, _N2),
        f2,
        f,
        y2.reshape(_N2, 1),
        y1.reshape(_N1, 1),
    )

    rows = pl.kernel(
        _sc_hist_body,
        out_type=jax.ShapeDtypeStruct((_NTILE * _SC_LANES,), jnp.float32),
        mesh=plsc.VectorSubcoreMesh(core_axis_name="c", subcore_axis_name="s"),
        scratch_types=[],
        compiler_params=_sc_compiler_params(),
    )(y1, y2)

    res = pl.pallas_call(
        _tc_combine_body,
        in_specs=[
            pl.BlockSpec(memory_space=pltpu.SMEM),
            pl.BlockSpec(memory_space=pltpu.VMEM),
        ],
        out_specs=pl.BlockSpec(memory_space=pltpu.SMEM),
        out_shape=jax.ShapeDtypeStruct((1,), jnp.float32),
    )(parts, rows.reshape(_NTILE, _SC_LANES))

    return res[0]


# speedup vs baseline: 1.0039x; 1.0025x over previous
import dataclasses

import jax
import jax.numpy as jnp
from jax import lax
from jax.experimental import pallas as pl
from jax.experimental.pallas import tpu as pltpu
from jax.experimental.pallas import tpu_sc as plsc

_N1 = 8192
_N2 = 8192
_C = 16
_R = 256
_NBLK = _N1 // _R

_SC_SUBCORES = 16
_SC_LANES = 16
_SEG = _N2 // _SC_SUBCORES


def _sc_compiler_params():
    cp = pltpu.CompilerParams()
    if "needs_layout_passes" in pltpu.CompilerParams.__dataclass_fields__:
        cp = dataclasses.replace(cp, needs_layout_passes=False)
    return cp


def _tc_body(y2r_ref, f2_ref, f_ref, y2c_ref, y1c_ref, out_ref, acc_ref):
    i = pl.program_id(0)
    _CW = 512

    def chunked_sums(x_ref, rowlab):
        parts_m = []
        parts_t = []
        for c in range(_N2 // _CW):
            sl = slice(c * _CW, (c + 1) * _CW)
            xc = x_ref[:, sl]
            mc = rowlab == y2r_ref[:, sl]
            parts_m.append(jnp.sum(jnp.where(mc, xc, 0.0), axis=0, keepdims=True))
            parts_t.append(jnp.sum(xc, axis=0, keepdims=True))
        return (jnp.concatenate(parts_m, axis=1),
                jnp.concatenate(parts_t, axis=1))

    s_m1, s_t1 = chunked_sums(f2_ref, y2c_ref[...])
    s_m2, s_t2 = chunked_sums(f_ref, y1c_ref[...])

    @pl.when(i == 0)
    def _init():
        acc_ref[0] = s_m1
        acc_ref[1] = s_t1
        acc_ref[2] = s_m2
        acc_ref[3] = s_t2

    @pl.when(i > 0)
    def _accum():
        acc_ref[0] += s_m1
        acc_ref[1] += s_t1
        acc_ref[2] += s_m2
        acc_ref[3] += s_t2

    @pl.when(i == _NBLK - 1)
    def _finalize():
        out_ref[0] = jnp.sum(acc_ref[0])
        out_ref[1] = jnp.sum(acc_ref[1])
        out_ref[2] = jnp.sum(acc_ref[2])
        out_ref[3] = jnp.sum(acc_ref[3])
        for k in range(4, 16):
            out_ref[k] = 0.0


_NTILE = 32


def _sc_hist_body(y1_hbm, y2_hbm, out_hbm):
    def tile_body(in_vmem, out_vmem):
        iota = lax.iota(jnp.int32, _SC_LANES)
        accs = tuple(jnp.zeros((_SC_LANES,), jnp.float32) for _ in range(_C))

        def chunk(j, accs):
            v = in_vmem[pl.ds(j * _SC_LANES, _SC_LANES)]
            return tuple(
                acc + jnp.where(v == c, 1.0, 0.0)
                for c, acc in enumerate(accs)
            )

        accs = lax.fori_loop(0, _SEG // _SC_LANES, chunk, accs)
        h = jnp.zeros((_SC_LANES,), jnp.float32)
        for c in range(_C):
            h = jnp.where(iota == c, jnp.sum(accs[c]), h)
        out_vmem[...] = h

    for idx, src in enumerate((y1_hbm, y2_hbm)):
        pltpu.emit_pipeline(
            tile_body,
            grid=(_SC_SUBCORES,),
            in_specs=[pl.BlockSpec((_SEG,), lambda i: (i,))],
            out_specs=[
                pl.BlockSpec(
                    (_SC_LANES,),
                    lambda i, idx=idx: (idx * _SC_SUBCORES + i,),
                )
            ],
            core_axis_name=("c", "s"),
            dimension_semantics=(pltpu.PARALLEL,),
        )(src, out_hbm)


def _tc_combine_body(parts_ref, rows_ref, out_ref):
    h1 = jnp.sum(rows_ref[0:_SC_SUBCORES, :], axis=0)
    h2 = jnp.sum(rows_ref[_SC_SUBCORES:_NTILE, :], axis=0)
    c22 = jnp.sum(h2 * h2)
    c12 = jnp.sum(h1 * h2)
    m1 = parts_ref[0]
    t1 = parts_ref[1]
    m2 = parts_ref[2]
    t2 = parts_ref[3]
    n1 = t1 - m1
    n2 = t2 - m2
    same1 = c22 - jnp.float32(_N2)
    different1 = jnp.float32(_N2) * jnp.float32(_N2) - c22
    same2 = c12
    different2 = jnp.float32(_N1) * jnp.float32(_N2) - c12
    out_ref[0] = (m1 / same1 + m2 / same2) / (
        n1 / different1 + n2 / different2
    )


def kernel(y1, y2, f, f2):
    y1 = y1.astype(jnp.int32)
    y2 = y2.astype(jnp.int32)

    rows = pl.kernel(
        _sc_hist_body,
        out_type=jax.ShapeDtypeStruct((_NTILE * _SC_LANES,), jnp.float32),
        mesh=plsc.VectorSubcoreMesh(core_axis_name="c", subcore_axis_name="s"),
        scratch_types=[],
        compiler_params=_sc_compiler_params(),
    )(y1, y2)

    parts = pl.pallas_call(
        _tc_body,
        grid=(_NBLK,),
        in_specs=[
            pl.BlockSpec((1, _N2), lambda i: (0, 0)),
            pl.BlockSpec((_R, _N2), lambda i: (i, 0)),
            pl.BlockSpec((_R, _N2), lambda i: (i, 0)),
            pl.BlockSpec((_R, 1), lambda i: (i, 0)),
            pl.BlockSpec((_R, 1), lambda i: (i, 0)),
        ],
        out_specs=pl.BlockSpec(memory_space=pltpu.SMEM),
        out_shape=jax.ShapeDtypeStruct((16,), jnp.float32),
        scratch_shapes=[pltpu.VMEM((4, 1, _N2), jnp.float32)],
        compiler_params=pltpu.CompilerParams(
            dimension_semantics=("arbitrary",),
        ),
    )(
        y2.reshape(1, _N2),
        f2,
        f,
        y2.reshape(_N2, 1),
        y1.reshape(_N1, 1),
    )

    res = pl.pallas_call(
        _tc_combine_body,
        in_specs=[
            pl.BlockSpec(memory_space=pltpu.SMEM),
            pl.BlockSpec(memory_space=pltpu.VMEM),
        ],
        out_specs=pl.BlockSpec(memory_space=pltpu.SMEM),
        out_shape=jax.ShapeDtypeStruct((1,), jnp.float32),
    )(parts, rows.reshape(_NTILE, _SC_LANES))

    return res[0]


# speedup vs baseline: 1.0138x; 1.0099x over previous
import dataclasses

import jax
import jax.numpy as jnp
from jax import lax
from jax.experimental import pallas as pl
from jax.experimental.pallas import tpu as pltpu
from jax.experimental.pallas import tpu_sc as plsc

_N1 = 8192
_N2 = 8192
_C = 16
_R = 256
_NBLK = _N1 // _R

_SC_SUBCORES = 16
_SC_LANES = 16
_SEG = _N2 // _SC_SUBCORES


def _sc_compiler_params():
    cp = pltpu.CompilerParams()
    if "needs_layout_passes" in pltpu.CompilerParams.__dataclass_fields__:
        cp = dataclasses.replace(cp, needs_layout_passes=False)
    return cp


def _tc_body(y2r_ref, f2_ref, f_ref, y2c_ref, y1c_ref, rows_ref, out_ref,
             acc_ref):
    i = pl.program_id(0)
    _CW = 512

    def chunked_sums(x_ref, rowlab):
        parts_m = []
        parts_t = []
        for c in range(_N2 // _CW):
            sl = slice(c * _CW, (c + 1) * _CW)
            xc = x_ref[:, sl]
            mc = rowlab == y2r_ref[:, sl]
            parts_m.append(jnp.sum(jnp.where(mc, xc, 0.0), axis=0, keepdims=True))
            parts_t.append(jnp.sum(xc, axis=0, keepdims=True))
        return (jnp.concatenate(parts_m, axis=1),
                jnp.concatenate(parts_t, axis=1))

    s_m1, s_t1 = chunked_sums(f2_ref, y2c_ref[...])
    s_m2, s_t2 = chunked_sums(f_ref, y1c_ref[...])

    @pl.when(i == 0)
    def _init():
        acc_ref[0] = s_m1
        acc_ref[1] = s_t1
        acc_ref[2] = s_m2
        acc_ref[3] = s_t2

    @pl.when(i > 0)
    def _accum():
        acc_ref[0] += s_m1
        acc_ref[1] += s_t1
        acc_ref[2] += s_m2
        acc_ref[3] += s_t2

    @pl.when(i == _NBLK - 1)
    def _finalize():
        h1 = jnp.sum(rows_ref[0:_SC_SUBCORES, :], axis=0)
        h2 = jnp.sum(rows_ref[_SC_SUBCORES:_NTILE, :], axis=0)
        c22 = jnp.sum(h2 * h2)
        c12 = jnp.sum(h1 * h2)
        m1 = jnp.sum(acc_ref[0])
        t1 = jnp.sum(acc_ref[1])
        m2 = jnp.sum(acc_ref[2])
        t2 = jnp.sum(acc_ref[3])
        n1 = t1 - m1
        n2 = t2 - m2
        same1 = c22 - jnp.float32(_N2)
        different1 = jnp.float32(_N2) * jnp.float32(_N2) - c22
        same2 = c12
        different2 = jnp.float32(_N1) * jnp.float32(_N2) - c12
        out_ref[0] = (m1 / same1 + m2 / same2) / (
            n1 / different1 + n2 / different2
        )


_NTILE = 32


def _sc_hist_body(y1_hbm, y2_hbm, out_hbm):
    def tile_body(in_vmem, out_vmem):
        iota = lax.iota(jnp.int32, _SC_LANES)
        accs = tuple(jnp.zeros((_SC_LANES,), jnp.float32) for _ in range(_C))

        def chunk(j, accs):
            v = in_vmem[pl.ds(j * _SC_LANES, _SC_LANES)]
            return tuple(
                acc + jnp.where(v == c, 1.0, 0.0)
                for c, acc in enumerate(accs)
            )

        accs = lax.fori_loop(0, _SEG // _SC_LANES, chunk, accs)
        h = jnp.zeros((_SC_LANES,), jnp.float32)
        for c in range(_C):
            h = jnp.where(iota == c, jnp.sum(accs[c]), h)
        out_vmem[...] = h

    for idx, src in enumerate((y1_hbm, y2_hbm)):
        pltpu.emit_pipeline(
            tile_body,
            grid=(_SC_SUBCORES,),
            in_specs=[pl.BlockSpec((_SEG,), lambda i: (i,))],
            out_specs=[
                pl.BlockSpec(
                    (_SC_LANES,),
                    lambda i, idx=idx: (idx * _SC_SUBCORES + i,),
                )
            ],
            core_axis_name=("c", "s"),
            dimension_semantics=(pltpu.PARALLEL,),
        )(src, out_hbm)


def kernel(y1, y2, f, f2):
    y1 = y1.astype(jnp.int32)
    y2 = y2.astype(jnp.int32)

    rows = pl.kernel(
        _sc_hist_body,
        out_type=jax.ShapeDtypeStruct((_NTILE * _SC_LANES,), jnp.float32),
        mesh=plsc.VectorSubcoreMesh(core_axis_name="c", subcore_axis_name="s"),
        scratch_types=[],
        compiler_params=_sc_compiler_params(),
    )(y1, y2)

    res = pl.pallas_call(
        _tc_body,
        grid=(_NBLK,),
        in_specs=[
            pl.BlockSpec((1, _N2), lambda i: (0, 0)),
            pl.BlockSpec((_R, _N2), lambda i: (i, 0)),
            pl.BlockSpec((_R, _N2), lambda i: (i, 0)),
            pl.BlockSpec((_R, 1), lambda i: (i, 0)),
            pl.BlockSpec((_R, 1), lambda i: (i, 0)),
            pl.BlockSpec((_NTILE, _SC_LANES), lambda i: (0, 0)),
        ],
        out_specs=pl.BlockSpec(memory_space=pltpu.SMEM),
        out_shape=jax.ShapeDtypeStruct((1,), jnp.float32),
        scratch_shapes=[pltpu.VMEM((4, 1, _N2), jnp.float32)],
        compiler_params=pltpu.CompilerParams(
            dimension_semantics=("arbitrary",),
        ),
    )(
        y2.reshape(1, _N2),
        f2,
        f,
        y2.reshape(_N2, 1),
        y1.reshape(_N1, 1),
        rows.reshape(_NTILE, _SC_LANES),
    )

    return res[0]
